# baseline (speedup 1.0000x reference)
import functools

import jax
import jax.numpy as jnp
from jax import lax
from jax.experimental import pallas as pl
from jax.experimental.pallas import tpu as pltpu
from jax.experimental.pallas import tpu_sc as plsc

_NC = 2
_NS = 16
_L = 16
_NW = _NC * _NS


@functools.lru_cache(maxsize=None)
def _build_sc_call(B, T, V):
    BT = B * T
    CHUNK = BT // _NW
    NCH = CHUNK // _L
    NG = CHUNK // 128
    assert BT % _NW == 0 and CHUNK % 128 == 0 and T % CHUNK == 0
    assert B == _L
    ACT_LOAD = CHUNK + _L

    mesh = plsc.VectorSubcoreMesh(core_axis_name="c", subcore_axis_name="s")

    @functools.partial(
        pl.kernel,
        mesh=mesh,
        out_type=jax.ShapeDtypeStruct((_NW, _L), jnp.float32),
        scratch_types=[
            pltpu.VMEM((ACT_LOAD,), jnp.int32),
            pltpu.VMEM((NG, 128), jnp.int32),
            pltpu.VMEM((NG, 128), jnp.int32),
            pltpu.VMEM((NG, 128), jnp.float32),
            pltpu.VMEM((NG, 128), jnp.float32),
            pltpu.VMEM((_L,), jnp.int32),
            pltpu.VMEM((_L,), jnp.float32),
            pltpu.VMEM((_L,), jnp.int32),
            pltpu.VMEM((_L,), jnp.int32),
            pltpu.VMEM((_L,), jnp.int32),
            pltpu.VMEM((_L,), jnp.int32),
            pltpu.VMEM((_L,), jnp.float32),
            pltpu.VMEM((_L,), jnp.float32),
            pltpu.VMEM((_L,), jnp.float32),
            pltpu.SemaphoreType.DMA,
        ],
    )
    def sc_call(logits_hbm, tgt_hbm, act_hbm, sl_hbm, rw_hbm, out_hbm,
                act_v, idx_q, idx_t, q_v, qt_v, sl_v, rw_v,
                cpos_v, cpos1_v, ca1_v, ca2_v, cq_v, cqt_v,
                part_v, sem):
        cid = lax.axis_index("c")
        sid = lax.axis_index("s")
        wid = sid * _NC + cid
        base = wid * CHUNK

        pltpu.sync_copy(act_hbm.at[pl.ds(base, ACT_LOAD)], act_v)

        iota = lax.iota(jnp.int32, _L)

        for j in range(NCH):
            a_q = act_v[pl.ds(j * _L, _L)]
            p_q = base + j * _L + iota
            idx_q[j // 8, pl.ds((j % 8) * _L, _L)] = p_q * V + a_q
            a_t = act_v[pl.ds(j * _L + 1, _L)]
            p_t = jnp.minimum(p_q + 1, BT - 1)
            idx_t[j // 8, pl.ds((j % 8) * _L, _L)] = p_t * V + a_t

        copies = []
        for c in range(NG):
            copies.append(pltpu.async_copy(logits_hbm.at[idx_q.at[c]],
                                           q_v.at[c], sem))
            copies.append(pltpu.async_copy(tgt_hbm.at[idx_t.at[c]],
                                           qt_v.at[c], sem))
        for cp in copies:
            cp.wait()

        t0 = base - (base // T) * T
        acc = jnp.zeros((_L,), jnp.float32)
        for j in range(NCH):
            q16 = q_v[j // 8, pl.ds((j % 8) * _L, _L)]
            qt16 = qt_v[j // 8, pl.ds((j % 8) * _L, _L)]
            qb = jnp.minimum(jnp.maximum(qt16, jnp.float32(-1.0)),
                             jnp.float32(0.0))
            if j == NCH - 1:
                t16 = t0 + j * _L + iota
                qb = jnp.where(t16 == T - 1, jnp.float32(0.0), qb)
            d = q16 - qb
            acc = acc + d * d

        part_v[...] = acc

        @pl.when(jnp.logical_and(cid == 0, sid == 0))
        def _():
            pltpu.sync_copy(sl_hbm, sl_v)
            pltpu.sync_copy(rw_hbm, rw_v)
            sl = sl_v[...]
            rw = rw_v[...]
            tpos = jnp.where(sl == 0, T - 1, sl - 1)
            pos = iota * T + tpos
            pos1 = jnp.minimum(pos + 1, BT - 1)
            cpos_v[...] = pos
            cpos1_v[...] = pos1
            pltpu.async_copy(act_hbm.at[cpos_v], ca1_v, sem).wait()
            pltpu.async_copy(act_hbm.at[cpos1_v], ca2_v, sem).wait()
            cpos_v[...] = pos * V + ca1_v[...]
            cpos1_v[...] = pos1 * V + ca2_v[...]
            pltpu.async_copy(logits_hbm.at[cpos_v], cq_v, sem).wait()
            pltpu.async_copy(tgt_hbm.at[cpos1_v], cqt_v, sem).wait()
            q = cq_v[...]
            qb0 = jnp.where(tpos == T - 1, jnp.float32(0.0), cqt_v[...])
            qb0 = jnp.minimum(jnp.maximum(qb0, jnp.float32(-1.0)),
                              jnp.float32(0.0))
            rr = jnp.minimum(jnp.maximum(rw, jnp.float32(-1.0)),
                             jnp.float32(0.0))
            d_new = q - rr
            d_old = q - qb0
            part_v[...] = part_v[...] + (d_new * d_new - d_old * d_old)

        pltpu.sync_copy(part_v, out_hbm.at[wid])

    return sc_call


def kernel(logits, tgt_logits, actions, rewards, seq_lens):
    B, T, V = logits.shape
    act_flat = jnp.pad(actions.reshape(-1).astype(jnp.int32), (0, 2 * _L))
    sc_call = _build_sc_call(B, T, V)
    partials = sc_call(
        logits.reshape(-1),
        tgt_logits.reshape(-1),
        act_flat,
        seq_lens.astype(jnp.int32),
        rewards.astype(jnp.float32),
    )
    return jnp.sum(partials)


# speedup vs baseline: 1.7168x; 1.7168x over previous
import functools

import jax
import jax.numpy as jnp
from jax import lax
from jax.experimental import pallas as pl
from jax.experimental.pallas import tpu as pltpu
from jax.experimental.pallas import tpu_sc as plsc

_NC = 2
_NS = 16
_L = 16
_NW = _NC * _NS
_WIN = 128
_BLK = 128


@functools.lru_cache(maxsize=None)
def _build_sc_call(B, T, V):
    BT = B * T
    CHUNK = BT // _NW
    NCH = CHUNK // _L
    NWIN = (V + _WIN - 1) // _WIN
    CAP = CHUNK
    assert BT % _NW == 0 and CHUNK % _BLK == 0 and T % CHUNK == 0
    assert B <= _L
    ACT_LOAD = CHUNK + _L

    mesh = plsc.VectorSubcoreMesh(core_axis_name="c", subcore_axis_name="s")

    @functools.partial(
        pl.kernel,
        mesh=mesh,
        compiler_params=pltpu.CompilerParams(needs_layout_passes=False),
        out_type=jax.ShapeDtypeStruct((_NW, _L), jnp.float32),
        scratch_types=[
            pltpu.VMEM((ACT_LOAD,), jnp.int32),
            pltpu.VMEM((NWIN * CAP,), jnp.int32),
            pltpu.VMEM((NWIN * CAP,), jnp.int32),
            pltpu.VMEM((NWIN * CAP,), jnp.int32),
            pltpu.VMEM((NWIN * CAP,), jnp.int32),
            pltpu.VMEM((NWIN * CAP,), jnp.int32),
            pltpu.VMEM((NWIN * CAP,), jnp.int32),
            pltpu.VMEM((_L,), jnp.int32),
            pltpu.VMEM((_L,), jnp.int32),
            pltpu.VMEM((_BLK, _WIN), jnp.float32),
            pltpu.VMEM((_BLK, _WIN), jnp.float32),
            pltpu.VMEM((CHUNK,), jnp.float32),
            pltpu.VMEM((CHUNK,), jnp.float32),
            pltpu.VMEM((_L,), jnp.int32),
            pltpu.VMEM((_L,), jnp.float32),
            pltpu.VMEM((_L,), jnp.float32),
            pltpu.SemaphoreType.DMA,
            pltpu.SemaphoreType.DMA,
        ],
    )
    def sc_call(matq_hbm, matt_hbm, tailq_hbm, tailt_hbm,
                act_hbm, sl_hbm, rw_hbm, out_hbm,
                act_v, qbidx_v, qboff_v, qbdst_v, tbidx_v, tboff_v, tbdst_v,
                cntq_v, cntt_v, gA_v, gB_v, q_v, qt_v, sl_v, rw_v, part_v,
                semA, semB):
        cid = lax.axis_index("c")
        sid = lax.axis_index("s")
        wid = sid * _NC + cid
        base = wid * CHUNK
        iota = lax.iota(jnp.int32, _L)

        pltpu.sync_copy(act_hbm.at[pl.ds(base, ACT_LOAD)], act_v)
        pltpu.sync_copy(sl_hbm, sl_v)
        pltpu.sync_copy(rw_hbm, rw_v)

        def prefill(i, _):
            rows = jnp.bitwise_and(base + i * _L + iota, BT - 1)
            qbidx_v[pl.ds(i * _L, _L)] = rows
            tbidx_v[pl.ds(i * _L, _L)] = rows
            return 0
        lax.fori_loop(0, (NWIN * CAP) // _L, prefill, 0)

        def mk_bucket(bidx_v, boff_v, bdst_v, cnt_v, ashift, clamp):
            def body(j, _):
                a16 = act_v[pl.ds(j * _L + ashift, _L)]
                d16 = j * _L + iota
                p16 = base + d16 + ashift
                if clamp:
                    p16 = jnp.minimum(p16, BT - 1)
                c16 = lax.shift_right_logical(a16, _WIN.bit_length() - 1)
                o16 = jnp.bitwise_and(a16, _WIN - 1)
                basec = plsc.load_gather(cnt_v, [c16])
                rank, lastm = plsc.scan_count(c16)
                slot = c16 * CAP + basec + rank - 1
                plsc.store_scatter(bidx_v, [slot], p16)
                plsc.store_scatter(boff_v, [slot], o16)
                plsc.store_scatter(bdst_v, [slot], d16)
                plsc.store_scatter(cnt_v, [c16], basec + rank, mask=lastm)
                return 0
            cnt_v[...] = jnp.zeros((_L,), jnp.int32)
            lax.fori_loop(0, NCH, body, 0)

        mk_bucket(qbidx_v, qboff_v, qbdst_v, cntq_v, 0, False)
        mk_bucket(tbidx_v, tboff_v, tbdst_v, cntt_v, 1, True)
        cntsq = cntq_v[...]
        cntst = cntt_v[...]

        TAIL_ADJ = (_WIN - V % _WIN) % _WIN
        assert NWIN % 2 == 0 and TAIL_ADJ

        def gather_tensor(mat, tail, bidx_v, boff_v, bdst_v, cnt_ref, out_v):

            def n_of(c):
                return plsc.load_gather(cnt_ref, [iota * 0 + c])

            def fire_blk(src, start, c, k, g_v, sem):
                idxref = bidx_v.at[pl.ds(c * CAP + k * _BLK, _BLK)]
                return pltpu.async_copy(
                    src.at[idxref, pl.ds(start, _WIN)], g_v, sem)

            def picks(g_v, c, k, nvec, adj):
                for m in range(_BLK // _L):
                    pos0 = k * _BLK + m * _L
                    offs = jnp.bitwise_and(
                        boff_v[pl.ds(c * CAP + pos0, _L)] + adj, _WIN - 1)
                    dsts = bdst_v[pl.ds(c * CAP + pos0, _L)]
                    valid = (pos0 + iota) < nvec
                    picked = plsc.load_gather(g_v, [m * _L + iota, offs],
                                              mask=valid)
                    plsc.store_scatter(out_v, [dsts], picked, mask=valid)

            def overflow(src, start, c, nvec, adj):
                nblk = lax.div(nvec[0] + (_BLK - 1), _BLK)

                def more(k, _):
                    fire_blk(src, start, c, k, gA_v, semA).wait()
                    picks(gA_v, c, k, nvec, adj)
                    return 0
                lax.fori_loop(1, nblk, more, 0)

            def window(src, start, c, k, g_v, sem_h, adj):
                sem_h.wait()
                nvec = n_of(c)
                picks(g_v, c, k, nvec, adj)
                overflow(src, start, c, nvec, adj)

            def pair_body(i, _):
                c0 = i * 2
                c1 = c0 + 1
                hA = fire_blk(mat, c0 * _WIN, c0, 0, gA_v, semA)
                hB = fire_blk(mat, c1 * _WIN, c1, 0, gB_v, semB)
                window(mat, c0 * _WIN, c0, 0, gA_v, hA, 0)
                window(mat, c1 * _WIN, c1, 0, gB_v, hB, 0)
                return 0

            lax.fori_loop(0, NWIN // 2 - 1, pair_body, 0)
            c0 = NWIN - 2
            c1 = NWIN - 1
            hA = fire_blk(mat, c0 * _WIN, c0, 0, gA_v, semA)
            hB = fire_blk(tail, 0, c1, 0, gB_v, semB)
            window(mat, c0 * _WIN, c0, 0, gA_v, hA, 0)
            window(tail, 0, c1, 0, gB_v, hB, TAIL_ADJ)

        gather_tensor(matq_hbm, tailq_hbm, qbidx_v, qboff_v, qbdst_v,
                      cntq_v, q_v)
        gather_tensor(matt_hbm, tailt_hbm, tbidx_v, tboff_v, tbdst_v,
                      cntt_v, qt_v)

        row = base // T
        t0 = base - row * T
        row16 = iota * 0 + row
        slr = plsc.load_gather(sl_v, [row16])
        rwr = plsc.load_gather(rw_v, [row16])
        tposv = jnp.where(slr == 0, T - 1, slr - 1)

        def acc_body(j, acc):
            q16 = q_v[pl.ds(j * _L, _L)]
            qt16 = qt_v[pl.ds(j * _L, _L)]
            t16 = t0 + j * _L + iota
            qb = jnp.where(t16 == T - 1, jnp.float32(0.0), qt16)
            qb = jnp.where(t16 == tposv, rwr, qb)
            qb = jnp.minimum(jnp.maximum(qb, jnp.float32(-1.0)),
                             jnp.float32(0.0))
            d = q16 - qb
            return acc + d * d

        acc = lax.fori_loop(0, NCH, acc_body, jnp.zeros((_L,), jnp.float32))
        part_v[...] = acc
        pltpu.sync_copy(part_v, out_hbm.at[wid])

    return sc_call


def kernel(logits, tgt_logits, actions, rewards, seq_lens):
    B, T, V = logits.shape
    act_flat = jnp.pad(actions.reshape(-1).astype(jnp.int32), (0, 2 * _L))
    mq = logits.reshape(B * T, V)
    mt = tgt_logits.reshape(B * T, V)
    sc_call = _build_sc_call(B, T, V)
    partials = sc_call(
        mq,
        mt,
        mq[:, V - _WIN:],
        mt[:, V - _WIN:],
        act_flat,
        seq_lens.astype(jnp.int32),
        rewards.astype(jnp.float32),
    )
    return jnp.sum(partials)


# speedup vs baseline: 10.2379x; 5.9635x over previous
import functools

import jax
import jax.numpy as jnp
from jax import lax
from jax.experimental import pallas as pl
from jax.experimental.pallas import tpu as pltpu
from jax.experimental.pallas import tpu_sc as plsc

_NC = 2
_NS = 16
_L = 16
_NW = _NC * _NS
_WIN = 128


@functools.lru_cache(maxsize=None)
def _build_sc_call(B, T, V):
    BT = B * T
    CHUNK = BT // _NW
    NCH = CHUNK // _L
    NWIN = CHUNK // _WIN
    assert BT % _NW == 0 and CHUNK % _WIN == 0 and T % CHUNK == 0
    assert T % _WIN == 0 and B <= _L
    ACT_LOAD = CHUNK + _L

    mesh = plsc.VectorSubcoreMesh(core_axis_name="c", subcore_axis_name="s")

    @functools.partial(
        pl.kernel,
        mesh=mesh,
        compiler_params=pltpu.CompilerParams(needs_layout_passes=False),
        out_type=jax.ShapeDtypeStruct((_NW, _L), jnp.float32),
        scratch_types=[
            pltpu.VMEM((ACT_LOAD,), jnp.int32),
            pltpu.VMEM((CHUNK,), jnp.int32),
            pltpu.VMEM((_WIN, _WIN), jnp.float32),
            pltpu.VMEM((_WIN, _WIN), jnp.float32),
            pltpu.VMEM((_WIN, _WIN), jnp.float32),
            pltpu.VMEM((_WIN, _WIN), jnp.float32),
            pltpu.VMEM((_WIN,), jnp.float32),
            pltpu.VMEM((CHUNK + _L,), jnp.float32),
            pltpu.VMEM((CHUNK + _L,), jnp.float32),
            pltpu.VMEM((_L,), jnp.int32),
            pltpu.VMEM((_L,), jnp.float32),
            pltpu.VMEM((_L,), jnp.float32),
            pltpu.SemaphoreType.DMA,
            pltpu.SemaphoreType.DMA,
            pltpu.SemaphoreType.DMA,
            pltpu.SemaphoreType.DMA,
        ],
    )
    def sc_call(matq_hbm, matt_hbm, act_hbm, sl_hbm, rw_hbm, out_hbm,
                act_v, ridx_v, gqA_v, gtA_v, gqB_v, gtB_v, xtr_v,
                q_v, qt_v, sl_v, rw_v, part_v,
                semqA, semtA, semqB, semtB):
        cid = lax.axis_index("c")
        sid = lax.axis_index("s")
        wid = sid * _NC + cid
        base = wid * CHUNK
        row = base // T
        t0 = base - row * T
        iota = lax.iota(jnp.int32, _L)

        pltpu.sync_copy(act_hbm.at[pl.ds(base, ACT_LOAD)], act_v)
        pltpu.sync_copy(sl_hbm, sl_v)
        pltpu.sync_copy(rw_hbm, rw_v)

        rowbase = row * V

        def mk_ridx(j, _):
            a16 = act_v[pl.ds(j * _L, _L)]
            ridx_v[pl.ds(j * _L, _L)] = rowbase + a16
            return 0
        lax.fori_loop(0, NCH, mk_ridx, 0)

        bufs = [(gqA_v, semqA, gtA_v, semtA), (gqB_v, semqB, gtB_v, semtB)]

        def fire(w, buf):
            gq, sq, gt, st = buf
            idxref = ridx_v.at[pl.ds(w * _WIN, _WIN)]
            col = t0 + w * _WIN
            hq = pltpu.async_copy(matq_hbm.at[idxref, pl.ds(col, _WIN)],
                                  gq, sq)
            ht = pltpu.async_copy(matt_hbm.at[idxref, pl.ds(col, _WIN)],
                                  gt, st)
            return hq, ht

        handles = [None, None]
        handles[0] = fire(0, bufs[0])
        for w in range(NWIN):
            if w + 1 < NWIN:
                handles[(w + 1) % 2] = fire(w + 1, bufs[(w + 1) % 2])
            hq, ht = handles[w % 2]
            hq.wait()
            ht.wait()
            gq, _, gt, _ = bufs[w % 2]
            for m in range(_WIN // _L):
                ii = m * _L + iota
                q_v[pl.ds(w * _WIN + m * _L, _L)] = \
                    plsc.load_gather(gq, [ii, ii])
                qt_v[pl.ds(w * _WIN + m * _L, _L)] = \
                    plsc.load_gather(gt, [ii, ii])

        @pl.when(t0 == 0)
        def _():
            a_x = act_v[pl.ds(CHUNK, _L)][0]
            pltpu.sync_copy(
                matt_hbm.at[rowbase + a_x, pl.ds(t0 + CHUNK, _WIN)], xtr_v)
            plsc.store_scatter(qt_v, [iota * 0 + CHUNK],
                               xtr_v[pl.ds(0, _L)], mask=iota == 0)

        row16 = iota * 0 + row
        slr = plsc.load_gather(sl_v, [row16])
        rwr = plsc.load_gather(rw_v, [row16])
        tposv = jnp.where(slr == 0, T - 1, slr - 1)

        def acc_body(j, acc):
            q16 = q_v[pl.ds(j * _L, _L)]
            qtn16 = qt_v[pl.ds(j * _L + 1, _L)]
            t16 = t0 + j * _L + iota
            qb = jnp.where(t16 == T - 1, jnp.float32(0.0), qtn16)
            qb = jnp.where(t16 == tposv, rwr, qb)
            qb = jnp.minimum(jnp.maximum(qb, jnp.float32(-1.0)),
                             jnp.float32(0.0))
            d = q16 - qb
            return acc + d * d

        acc = lax.fori_loop(0, NCH, acc_body, jnp.zeros((_L,), jnp.float32))
        part_v[...] = acc
        pltpu.sync_copy(part_v, out_hbm.at[wid])

    return sc_call


def kernel(logits, tgt_logits, actions, rewards, seq_lens):
    B, T, V = logits.shape
    act_flat = jnp.pad(actions.reshape(-1).astype(jnp.int32), (0, 2 * _L))
    sc_call = _build_sc_call(B, T, V)
    partials = sc_call(
        logits.transpose(0, 2, 1).reshape(B * V, T),
        tgt_logits.transpose(0, 2, 1).reshape(B * V, T),
        act_flat,
        seq_lens.astype(jnp.int32),
        rewards.astype(jnp.float32),
    )
    return jnp.sum(partials)


# speedup vs baseline: 18.1927x; 1.7770x over previous
import functools

import jax
import jax.numpy as jnp
from jax import lax
from jax.experimental import pallas as pl
from jax.experimental.pallas import tpu as pltpu
from jax.experimental.pallas import tpu_sc as plsc

_NC = 2
_NS = 16
_L = 16
_NW = _NC * _NS


@functools.lru_cache(maxsize=None)
def _build_sc_call(B, T, V):
    BT = B * T
    CHUNK = BT // _NW
    NCH = CHUNK // _L
    NG = CHUNK // 128
    assert BT % _NW == 0 and CHUNK % 128 == 0 and T % CHUNK == 0
    assert V % 8 == 0 and T % 128 == 0 and B <= _L
    ACT_LOAD = CHUNK + _L
    TSH = (8 * T).bit_length() - 1

    mesh = plsc.VectorSubcoreMesh(core_axis_name="c", subcore_axis_name="s")

    @functools.partial(
        pl.kernel,
        mesh=mesh,
        compiler_params=pltpu.CompilerParams(needs_layout_passes=False),
        out_type=jax.ShapeDtypeStruct((_NW, _L), jnp.float32),
        scratch_types=[
            pltpu.VMEM((ACT_LOAD,), jnp.int32),
            pltpu.VMEM((NG, 128), jnp.int32),
            pltpu.VMEM((_L,), jnp.int32),
            pltpu.VMEM((CHUNK + _L,), jnp.float32),
            pltpu.VMEM((CHUNK + _L,), jnp.float32),
            pltpu.VMEM((_L,), jnp.float32),
            pltpu.VMEM((_L,), jnp.int32),
            pltpu.VMEM((_L,), jnp.float32),
            pltpu.VMEM((_L,), jnp.float32),
            pltpu.SemaphoreType.DMA,
            pltpu.SemaphoreType.DMA,
        ],
    )
    def sc_call(matq_hbm, matt_hbm, act_hbm, sl_hbm, rw_hbm, out_hbm,
                act_v, offs_v, offx_v, q_v, qt_v, xtr_v, sl_v, rw_v,
                part_v, semq, semt):
        cid = lax.axis_index("c")
        sid = lax.axis_index("s")
        wid = sid * _NC + cid
        base = wid * CHUNK
        row = base // T
        t0 = base - row * T
        iota = lax.iota(jnp.int32, _L)

        pltpu.sync_copy(act_hbm.at[pl.ds(base, ACT_LOAD)], act_v)
        pltpu.sync_copy(sl_hbm, sl_v)
        pltpu.sync_copy(rw_hbm, rw_v)

        rowbase = row * V

        def phys_off(t16, a16):
            r16 = rowbase + a16
            return (lax.shift_left(lax.shift_right_logical(r16, 3), TSH)
                    + lax.shift_left(lax.shift_right_logical(t16, 7), 10)
                    + lax.shift_left(jnp.bitwise_and(r16, 7), 7)
                    + jnp.bitwise_and(t16, 127))

        def mk_offs(j, _):
            a16 = act_v[pl.ds(j * _L, _L)]
            t16 = t0 + j * _L + iota
            offs_v[j // 8, pl.ds((j % 8) * _L, _L)] = phys_off(t16, a16)
            return 0
        lax.fori_loop(0, NCH, mk_offs, 0)

        a_x16 = act_v[pl.ds(CHUNK, _L)]
        t_x16 = jnp.minimum(t0 + CHUNK + iota, T - 1)
        offx_v[...] = phys_off(t_x16, a_x16)

        copies = []
        for c in range(NG):
            copies.append(pltpu.async_copy(
                matq_hbm.at[offs_v.at[c]],
                q_v.at[pl.ds(c * 128, 128)], semq))
            copies.append(pltpu.async_copy(
                matt_hbm.at[offs_v.at[c]],
                qt_v.at[pl.ds(c * 128, 128)], semt))
        copies.append(pltpu.async_copy(matt_hbm.at[offx_v], xtr_v, semt))
        for cp in copies:
            cp.wait()
        plsc.store_scatter(qt_v, [iota * 0 + CHUNK], xtr_v[...],
                           mask=iota == 0)

        row16 = iota * 0 + row
        slr = plsc.load_gather(sl_v, [row16])
        rwr = plsc.load_gather(rw_v, [row16])
        tposv = jnp.where(slr == 0, T - 1, slr - 1)

        def acc_body(j, acc):
            q16 = q_v[pl.ds(j * _L, _L)]
            qtn16 = qt_v[pl.ds(j * _L + 1, _L)]
            t16 = t0 + j * _L + iota
            qb = jnp.where(t16 == T - 1, jnp.float32(0.0), qtn16)
            qb = jnp.where(t16 == tposv, rwr, qb)
            qb = jnp.minimum(jnp.maximum(qb, jnp.float32(-1.0)),
                             jnp.float32(0.0))
            d = q16 - qb
            return acc + d * d

        acc = lax.fori_loop(0, NCH, acc_body, jnp.zeros((_L,), jnp.float32))
        part_v[...] = acc
        pltpu.sync_copy(part_v, out_hbm.at[wid])

    return sc_call


def _phys_flat(x, B, T, V):
    return (x.transpose(0, 2, 1)
            .reshape(B, V // 8, 8, T // 128, 128)
            .transpose(0, 1, 3, 2, 4)
            .reshape(-1))


def kernel(logits, tgt_logits, actions, rewards, seq_lens):
    B, T, V = logits.shape
    act_flat = jnp.pad(actions.reshape(-1).astype(jnp.int32), (0, 2 * _L))
    sc_call = _build_sc_call(B, T, V)
    partials = sc_call(
        _phys_flat(logits, B, T, V),
        _phys_flat(tgt_logits, B, T, V),
        act_flat,
        seq_lens.astype(jnp.int32),
        rewards.astype(jnp.float32),
    )
    return jnp.sum(partials)


# speedup vs baseline: 18.3651x; 1.0095x over previous
import functools

import jax
import jax.numpy as jnp
from jax import lax
from jax.experimental import pallas as pl
from jax.experimental.pallas import tpu as pltpu
from jax.experimental.pallas import tpu_sc as plsc

_NC = 2
_NS = 16
_L = 16
_NW = _NC * _NS


@functools.lru_cache(maxsize=None)
def _build_sc_call(B, T, V):
    BT = B * T
    CHUNK = BT // _NW
    NCH = CHUNK // _L
    NG = CHUNK // 128
    assert BT % _NW == 0 and CHUNK % 128 == 0 and T % CHUNK == 0
    assert V % 8 == 0 and T % 128 == 0 and B % 8 == 0 and B <= _L
    NAW = CHUNK // 128 + 1
    ACT_LOAD = NAW * 128
    NTW = T // 128
    TSH = (8 * T).bit_length() - 1

    mesh = plsc.VectorSubcoreMesh(core_axis_name="c", subcore_axis_name="s")

    @functools.partial(
        pl.kernel,
        mesh=mesh,
        compiler_params=pltpu.CompilerParams(needs_layout_passes=False),
        out_type=jax.ShapeDtypeStruct((_NW, _L), jnp.float32),
        scratch_types=[
            pltpu.VMEM((ACT_LOAD,), jnp.int32),
            pltpu.VMEM((NG, 128), jnp.int32),
            pltpu.VMEM((_L,), jnp.int32),
            pltpu.VMEM((CHUNK + _L,), jnp.float32),
            pltpu.VMEM((CHUNK + _L,), jnp.float32),
            pltpu.VMEM((_L,), jnp.float32),
            pltpu.VMEM((_L,), jnp.int32),
            pltpu.VMEM((_L,), jnp.float32),
            pltpu.VMEM((_L,), jnp.float32),
            pltpu.SemaphoreType.DMA,
            pltpu.SemaphoreType.DMA,
            pltpu.SemaphoreType.DMA,
        ],
    )
    def sc_call(matq_hbm, matt_hbm, act_hbm, sl_hbm, rw_hbm, out_hbm,
                act_v, offs_v, offx_v, q_v, qt_v, xtr_v, sl_v, rw_v,
                part_v, semq, semt, sema):
        cid = lax.axis_index("c")
        sid = lax.axis_index("s")
        wid = sid * _NC + cid
        base = wid * CHUNK
        row = base // T
        t0 = base - row * T
        iota = lax.iota(jnp.int32, _L)

        g8 = lax.shift_right_logical(row, 3)
        s8 = jnp.bitwise_and(row, 7)
        u0 = lax.shift_right_logical(t0, 7)
        acopies = []
        for k in range(NAW):
            u = jnp.minimum(u0 + k, NTW - 1)
            src = ((g8 * NTW + u) * 8 + s8) * 128
            acopies.append(pltpu.async_copy(
                act_hbm.at[pl.ds(src, 128)],
                act_v.at[pl.ds(k * 128, 128)], sema))
        pltpu.sync_copy(sl_hbm, sl_v)
        pltpu.sync_copy(rw_hbm, rw_v)
        for cp in acopies:
            cp.wait()

        rowbase = row * V

        def phys_off(t16, a16):
            r16 = rowbase + a16
            return (lax.shift_left(lax.shift_right_logical(r16, 3), TSH)
                    + lax.shift_left(lax.shift_right_logical(t16, 7), 10)
                    + lax.shift_left(jnp.bitwise_and(r16, 7), 7)
                    + jnp.bitwise_and(t16, 127))

        def mk_offs(j, _):
            a16 = act_v[pl.ds(j * _L, _L)]
            t16 = t0 + j * _L + iota
            offs_v[j // 8, pl.ds((j % 8) * _L, _L)] = phys_off(t16, a16)
            return 0
        lax.fori_loop(0, NCH, mk_offs, 0)

        a_x16 = act_v[pl.ds(CHUNK, _L)]
        t_x16 = jnp.minimum(t0 + CHUNK + iota, T - 1)
        offx_v[...] = phys_off(t_x16, a_x16)

        copies = []
        for c in range(NG):
            copies.append(pltpu.async_copy(
                matq_hbm.at[offs_v.at[c]],
                q_v.at[pl.ds(c * 128, 128)], semq))
            copies.append(pltpu.async_copy(
                matt_hbm.at[offs_v.at[c]],
                qt_v.at[pl.ds(c * 128, 128)], semt))
        copies.append(pltpu.async_copy(matt_hbm.at[offx_v], xtr_v, semt))
        for cp in copies:
            cp.wait()
        plsc.store_scatter(qt_v, [iota * 0 + CHUNK], xtr_v[...],
                           mask=iota == 0)

        row16 = iota * 0 + row
        slr = plsc.load_gather(sl_v, [row16])
        rwr = plsc.load_gather(rw_v, [row16])
        tposv = jnp.where(slr == 0, T - 1, slr - 1)

        def acc_step(j, acc, last):
            q16 = q_v[pl.ds(j * _L, _L)]
            qtn16 = qt_v[pl.ds(j * _L + 1, _L)]
            t16 = t0 + j * _L + iota
            qb = qtn16
            if last:
                qb = jnp.where(t16 == T - 1, jnp.float32(0.0), qb)
            qb = jnp.where(t16 == tposv, rwr, qb)
            qb = jnp.minimum(jnp.maximum(qb, jnp.float32(-1.0)),
                             jnp.float32(0.0))
            d = q16 - qb
            return acc + d * d

        acc = lax.fori_loop(
            0, NCH - 1, lambda j, a: acc_step(j, a, False),
            jnp.zeros((_L,), jnp.float32))
        acc = acc_step(NCH - 1, acc, True)
        part_v[...] = acc
        pltpu.sync_copy(part_v, out_hbm.at[wid])

    return sc_call


def _phys_flat(x, B, T, V):
    return (x.transpose(0, 2, 1)
            .reshape(B, V // 8, 8, T // 128, 128)
            .transpose(0, 1, 3, 2, 4)
            .reshape(-1))


def _phys_flat_2d(x, B, T):
    return (x.reshape(B // 8, 8, T // 128, 128)
            .transpose(0, 2, 1, 3)
            .reshape(-1))


def kernel(logits, tgt_logits, actions, rewards, seq_lens):
    B, T, V = logits.shape
    sc_call = _build_sc_call(B, T, V)
    partials = sc_call(
        _phys_flat(logits, B, T, V),
        _phys_flat(tgt_logits, B, T, V),
        _phys_flat_2d(actions.astype(jnp.int32), B, T),
        seq_lens.astype(jnp.int32),
        rewards.astype(jnp.float32),
    )
    return jnp.sum(partials)
